# initial kernel scaffold (unmeasured)
import jax
import jax.numpy as jnp
from jax import lax
from jax.experimental import pallas as pl
from jax.experimental.pallas import tpu as pltpu

M = 4096
K = 8192
N = 4096
MQ = M // 4
N_BLK = 512
K_BLK = 2048
N_STEPS = N // N_BLK
K_STEPS = K // K_BLK


def kernel(dy, W):
    my_x = lax.axis_index("x")
    my_y = lax.axis_index("y")
    q = 2 * my_x + my_y
    dy_q = lax.dynamic_slice(dy, (q * MQ, 0), (MQ, K))

    def body(dy_ref, w_ref, out_ref, c_ref, zrecv_ref,
             local_sem, z_send, z_recv, x_send, x_recv, y_send, y_recv):
        n = pl.program_id(0)
        k = pl.program_id(1)
        mx = lax.axis_index("x")
        my = lax.axis_index("y")
        mz = lax.axis_index("z")
        qq = 2 * mx + my

        @pl.when(jnp.logical_and(n == 0, k == 0))
        def _barrier():
            bsem = pltpu.get_barrier_semaphore()
            for nbr in ((1 - mx, my, mz), (mx, 1 - my, mz), (mx, my, 1 - mz)):
                pl.semaphore_signal(bsem, inc=1, device_id=nbr,
                                    device_id_type=pl.DeviceIdType.MESH)
            pl.semaphore_wait(bsem, 3)

        acc = lax.dot_general(
            dy_ref[...], w_ref[...],
            (((1,), (1,)), ((), ())),
            preferred_element_type=jnp.float32,
        )
        nsl = pl.ds(n * N_BLK, N_BLK)

        @pl.when(k == 0)
        def _init():
            c_ref[:, nsl] = acc

        @pl.when(k != 0)
        def _accum():
            c_ref[:, nsl] = c_ref[:, nsl] + acc

        @pl.when(jnp.logical_and(n == N_STEPS - 1, k == K_STEPS - 1))
        def _phases():
            rz = pltpu.make_async_remote_copy(
                src_ref=c_ref, dst_ref=zrecv_ref,
                send_sem=z_send, recv_sem=z_recv,
                device_id=(mx, my, 1 - mz),
                device_id_type=pl.DeviceIdType.MESH,
            )
            rz.start()
            rz.wait()
            c_ref[...] = c_ref[...] + zrecv_ref[...]

            rows_q = pl.ds(qq * MQ, MQ)
            cp = pltpu.make_async_copy(c_ref, out_ref.at[rows_q, :], local_sem)
            cp.start()
            cp.wait()

            rx = pltpu.make_async_remote_copy(
                src_ref=out_ref.at[rows_q, :], dst_ref=out_ref.at[rows_q, :],
                send_sem=x_send, recv_sem=x_recv,
                device_id=(1 - mx, my, mz),
                device_id_type=pl.DeviceIdType.MESH,
            )
            rx.start()
            rx.wait()

            q2 = 2 * (1 - mx) + my
            rys = []
            for i, qi in enumerate((qq, q2)):
                r = pl.ds(qi * MQ, MQ)
                ry = pltpu.make_async_remote_copy(
                    src_ref=out_ref.at[r, :], dst_ref=out_ref.at[r, :],
                    send_sem=y_send.at[i], recv_sem=y_recv.at[i],
                    device_id=(mx, 1 - my, mz),
                    device_id_type=pl.DeviceIdType.MESH,
                )
                ry.start()
                rys.append(ry)
            for ry in rys:
                ry.wait()

    return pl.pallas_call(
        body,
        grid=(N_STEPS, K_STEPS),
        in_specs=[
            pl.BlockSpec((MQ, K_BLK), lambda n, k: (0, k)),
            pl.BlockSpec((N_BLK, K_BLK), lambda n, k: (n, k)),
        ],
        out_specs=pl.BlockSpec(memory_space=pltpu.MemorySpace.ANY),
        out_shape=jax.ShapeDtypeStruct((M, N), jnp.float32),
        scratch_shapes=[
            pltpu.VMEM((MQ, N), jnp.float32),
            pltpu.VMEM((MQ, N), jnp.float32),
            pltpu.SemaphoreType.DMA,
            pltpu.SemaphoreType.DMA,
            pltpu.SemaphoreType.DMA,
            pltpu.SemaphoreType.DMA,
            pltpu.SemaphoreType.DMA,
            pltpu.SemaphoreType.DMA((2,)),
            pltpu.SemaphoreType.DMA((2,)),
        ],
        compiler_params=pltpu.CompilerParams(
            collective_id=0,
            dimension_semantics=("arbitrary", "arbitrary"),
        ),
    )(dy_q, W)


# baseline (device time: 933116 ns/iter reference)
import jax
import jax.numpy as jnp
from jax import lax
from jax.experimental import pallas as pl
from jax.experimental.pallas import tpu as pltpu

M = 4096
K = 8192
N = 4096
MQ = M // 4
N_BLK = 512
K_BLK = 2048
N_STEPS = N // N_BLK
K_STEPS = K // K_BLK


def kernel(dy, W):
    my_x = lax.axis_index("x")
    my_y = lax.axis_index("y")
    q = 2 * my_x + my_y
    dy_q = lax.dynamic_slice(dy, (q * MQ, 0), (MQ, K))

    def body(dy_ref, w_ref, out_ref, c_ref, zrecv_ref,
             local_sem, z_send, z_recv, x_send, x_recv, y_send, y_recv):
        n = pl.program_id(0)
        k = pl.program_id(1)
        mx = lax.axis_index("x")
        my = lax.axis_index("y")
        mz = lax.axis_index("z")
        qq = 2 * mx + my

        @pl.when(jnp.logical_and(n == 0, k == 0))
        def _barrier():
            bsem = pltpu.get_barrier_semaphore()
            for nbr in ((1 - mx, my, mz), (mx, 1 - my, mz), (mx, my, 1 - mz)):
                pl.semaphore_signal(bsem, inc=1, device_id=nbr,
                                    device_id_type=pl.DeviceIdType.MESH)
            pl.semaphore_wait(bsem, 3)

        acc = lax.dot_general(
            dy_ref[...], w_ref[...],
            (((1,), (1,)), ((), ())),
            preferred_element_type=jnp.float32,
        )
        nsl = pl.ds(n * N_BLK, N_BLK)

        @pl.when(k == 0)
        def _init():
            c_ref[:, nsl] = acc

        @pl.when(k != 0)
        def _accum():
            c_ref[:, nsl] = c_ref[:, nsl] + acc

        @pl.when(jnp.logical_and(n == N_STEPS - 1, k == K_STEPS - 1))
        def _phases():
            rz = pltpu.make_async_remote_copy(
                src_ref=c_ref, dst_ref=zrecv_ref,
                send_sem=z_send, recv_sem=z_recv,
                device_id=(mx, my, 1 - mz),
                device_id_type=pl.DeviceIdType.MESH,
            )
            rz.start()
            rz.wait()
            c_ref[...] = c_ref[...] + zrecv_ref[...]

            rows_q = pl.ds(qq * MQ, MQ)
            cp = pltpu.make_async_copy(c_ref, out_ref.at[rows_q, :], local_sem)
            cp.start()
            cp.wait()

            rx = pltpu.make_async_remote_copy(
                src_ref=out_ref.at[rows_q, :], dst_ref=out_ref.at[rows_q, :],
                send_sem=x_send, recv_sem=x_recv,
                device_id=(1 - mx, my, mz),
                device_id_type=pl.DeviceIdType.MESH,
            )
            rx.start()
            rx.wait()

            q2 = 2 * (1 - mx) + my
            rys = []
            for i, qi in enumerate((qq, q2)):
                r = pl.ds(qi * MQ, MQ)
                ry = pltpu.make_async_remote_copy(
                    src_ref=out_ref.at[r, :], dst_ref=out_ref.at[r, :],
                    send_sem=y_send.at[i], recv_sem=y_recv.at[i],
                    device_id=(mx, 1 - my, mz),
                    device_id_type=pl.DeviceIdType.MESH,
                )
                ry.start()
                rys.append(ry)
            for ry in rys:
                ry.wait()

    return pl.pallas_call(
        body,
        grid=(N_STEPS, K_STEPS),
        in_specs=[
            pl.BlockSpec((MQ, K_BLK), lambda n, k: (0, k)),
            pl.BlockSpec((N_BLK, K_BLK), lambda n, k: (n, k)),
        ],
        out_specs=pl.BlockSpec(memory_space=pl.ANY),
        out_shape=jax.ShapeDtypeStruct((M, N), jnp.float32),
        scratch_shapes=[
            pltpu.VMEM((MQ, N), jnp.float32),
            pltpu.VMEM((MQ, N), jnp.float32),
            pltpu.SemaphoreType.DMA,
            pltpu.SemaphoreType.DMA,
            pltpu.SemaphoreType.DMA,
            pltpu.SemaphoreType.DMA,
            pltpu.SemaphoreType.DMA,
            pltpu.SemaphoreType.DMA((2,)),
            pltpu.SemaphoreType.DMA((2,)),
        ],
        compiler_params=pltpu.CompilerParams(
            collective_id=0,
            dimension_semantics=("arbitrary", "arbitrary"),
            vmem_limit_bytes=64 * 1024 * 1024,
        ),
    )(dy_q, W)


# device time: 477563 ns/iter; 1.9539x vs baseline; 1.9539x over previous
import jax
import jax.numpy as jnp
from jax import lax
from jax.experimental import pallas as pl
from jax.experimental.pallas import tpu as pltpu

M = 4096
K = 8192
N = 4096
MQ = M // 4
N_BLK = 512
K_BLK = 2048
N_STEPS = N // N_BLK
K_STEPS = K // K_BLK


def kernel(dy, W):
    my_x = lax.axis_index("x")
    my_y = lax.axis_index("y")
    q = 2 * my_x + my_y
    dy_q = lax.dynamic_slice(dy, (q * MQ, 0), (MQ, K))

    def body(dy_ref, w_ref, out_ref, c_ref, zrecv_ref,
             local_sem, z_send, z_recv, xd_send, xd_recv,
             yd_send, yd_recv, yf_send, yf_recv):
        n = pl.program_id(0)
        k = pl.program_id(1)
        mx = lax.axis_index("x")
        my = lax.axis_index("y")
        mz = lax.axis_index("z")
        qq = 2 * mx + my
        q2 = 2 * (1 - mx) + my
        rows_q = pl.ds(qq * MQ, MQ)
        rows_q2 = pl.ds(q2 * MQ, MQ)

        def csl(m):
            return pl.ds(m * N_BLK, N_BLK)

        def z_descr(m):
            return pltpu.make_async_remote_copy(
                src_ref=c_ref.at[:, csl(m)], dst_ref=zrecv_ref.at[:, csl(m)],
                send_sem=z_send.at[m], recv_sem=z_recv.at[m],
                device_id=(mx, my, 1 - mz),
                device_id_type=pl.DeviceIdType.MESH,
            )

        def local_descr(m):
            return pltpu.make_async_copy(
                c_ref.at[:, csl(m)], out_ref.at[rows_q, csl(m)],
                local_sem.at[m],
            )

        def xd_descr(m):
            return pltpu.make_async_remote_copy(
                src_ref=c_ref.at[:, csl(m)], dst_ref=out_ref.at[rows_q, csl(m)],
                send_sem=xd_send.at[m], recv_sem=xd_recv.at[m],
                device_id=(1 - mx, my, mz),
                device_id_type=pl.DeviceIdType.MESH,
            )

        def yd_descr(m):
            return pltpu.make_async_remote_copy(
                src_ref=c_ref.at[:, csl(m)], dst_ref=out_ref.at[rows_q, csl(m)],
                send_sem=yd_send.at[m], recv_sem=yd_recv.at[m],
                device_id=(mx, 1 - my, mz),
                device_id_type=pl.DeviceIdType.MESH,
            )

        def yf_descr(m):
            return pltpu.make_async_remote_copy(
                src_ref=out_ref.at[rows_q2, csl(m)],
                dst_ref=out_ref.at[rows_q2, csl(m)],
                send_sem=yf_send.at[m], recv_sem=yf_recv.at[m],
                device_id=(mx, 1 - my, mz),
                device_id_type=pl.DeviceIdType.MESH,
            )

        def handle_z(m):
            z_descr(m).wait()
            sl = csl(m)
            c_ref[:, sl] = c_ref[:, sl] + zrecv_ref[:, sl]
            local_descr(m).start()
            xd_descr(m).start()
            yd_descr(m).start()

        def handle_xy(m):
            xd_descr(m).wait()
            yd_descr(m).wait()
            yf_descr(m).start()

        @pl.when(jnp.logical_and(n == 0, k == 0))
        def _barrier():
            bsem = pltpu.get_barrier_semaphore()
            for nbr in ((1 - mx, my, mz), (mx, 1 - my, mz), (mx, my, 1 - mz)):
                pl.semaphore_signal(bsem, inc=1, device_id=nbr,
                                    device_id_type=pl.DeviceIdType.MESH)
            pl.semaphore_wait(bsem, 3)

        acc = lax.dot_general(
            dy_ref[...], w_ref[...],
            (((1,), (1,)), ((), ())),
            preferred_element_type=jnp.float32,
        )
        nsl = csl(n)

        @pl.when(k == 0)
        def _init():
            c_ref[:, nsl] = acc

        @pl.when(k != 0)
        def _accum():
            c_ref[:, nsl] = c_ref[:, nsl] + acc

        @pl.when(k == K_STEPS - 1)
        def _comm():
            z_descr(n).start()

            @pl.when(n >= 1)
            def _():
                handle_z(n - 1)

            @pl.when(n >= 2)
            def _():
                handle_xy(n - 2)

            @pl.when(n == N_STEPS - 1)
            def _drain():
                handle_z(n)
                handle_xy(n - 1)
                handle_xy(n)
                for m in range(N_STEPS):
                    yf_descr(m).wait()
                    local_descr(m).wait()

    return pl.pallas_call(
        body,
        grid=(N_STEPS, K_STEPS),
        in_specs=[
            pl.BlockSpec((MQ, K_BLK), lambda n, k: (0, k)),
            pl.BlockSpec((N_BLK, K_BLK), lambda n, k: (n, k)),
        ],
        out_specs=pl.BlockSpec(memory_space=pl.ANY),
        out_shape=jax.ShapeDtypeStruct((M, N), jnp.float32),
        scratch_shapes=[
            pltpu.VMEM((MQ, N), jnp.float32),
            pltpu.VMEM((MQ, N), jnp.float32),
            pltpu.SemaphoreType.DMA((N_STEPS,)),
            pltpu.SemaphoreType.DMA((N_STEPS,)),
            pltpu.SemaphoreType.DMA((N_STEPS,)),
            pltpu.SemaphoreType.DMA((N_STEPS,)),
            pltpu.SemaphoreType.DMA((N_STEPS,)),
            pltpu.SemaphoreType.DMA((N_STEPS,)),
            pltpu.SemaphoreType.DMA((N_STEPS,)),
            pltpu.SemaphoreType.DMA((N_STEPS,)),
            pltpu.SemaphoreType.DMA((N_STEPS,)),
        ],
        compiler_params=pltpu.CompilerParams(
            collective_id=0,
            dimension_semantics=("arbitrary", "arbitrary"),
            vmem_limit_bytes=64 * 1024 * 1024,
        ),
    )(dy_q, W)


# device time: 389125 ns/iter; 2.3980x vs baseline; 1.2273x over previous
import jax
import jax.numpy as jnp
from jax import lax
from jax.experimental import pallas as pl
from jax.experimental.pallas import tpu as pltpu

M = 4096
K = 8192
N = 4096
MQ = M // 4
N_BLK = 512
K_BLK = 2048
N_STEPS = N // N_BLK
K_STEPS = K // K_BLK


def kernel(dy, W):
    my_x = lax.axis_index("x")
    my_y = lax.axis_index("y")
    q = 2 * my_x + my_y
    dy_q = lax.dynamic_slice(dy, (q * MQ, 0), (MQ, K))

    def body(dy_ref, w_ref, out_ref, c_ref, zrecv_ref,
             local_sem, z_send, z_recv, xd_send, xd_recv,
             yd_send, yd_recv, yf_send, yf_recv, xf_send, xf_recv):
        n = pl.program_id(0)
        k = pl.program_id(1)
        mx = lax.axis_index("x")
        my = lax.axis_index("y")
        mz = lax.axis_index("z")
        qq = 2 * mx + my
        q1 = 2 * mx + (1 - my)
        q2 = 2 * (1 - mx) + my
        rows_q = pl.ds(qq * MQ, MQ)
        rows_q1 = pl.ds(q1 * MQ, MQ)
        rows_q2 = pl.ds(q2 * MQ, MQ)
        HB = N_BLK // 2

        def csl(m):
            return pl.ds(m * N_BLK, N_BLK)

        def z_descr(m):
            return pltpu.make_async_remote_copy(
                src_ref=c_ref.at[:, csl(m)], dst_ref=zrecv_ref.at[:, csl(m)],
                send_sem=z_send.at[m], recv_sem=z_recv.at[m],
                device_id=(mx, my, 1 - mz),
                device_id_type=pl.DeviceIdType.MESH,
            )

        def local_descr(m):
            return pltpu.make_async_copy(
                c_ref.at[:, csl(m)], out_ref.at[rows_q, csl(m)],
                local_sem.at[m],
            )

        def xd_descr(m):
            return pltpu.make_async_remote_copy(
                src_ref=c_ref.at[:, csl(m)], dst_ref=out_ref.at[rows_q, csl(m)],
                send_sem=xd_send.at[m], recv_sem=xd_recv.at[m],
                device_id=(1 - mx, my, mz),
                device_id_type=pl.DeviceIdType.MESH,
            )

        def yd_descr(m):
            return pltpu.make_async_remote_copy(
                src_ref=c_ref.at[:, csl(m)], dst_ref=out_ref.at[rows_q, csl(m)],
                send_sem=yd_send.at[m], recv_sem=yd_recv.at[m],
                device_id=(mx, 1 - my, mz),
                device_id_type=pl.DeviceIdType.MESH,
            )

        def yf_descr(m):
            sl = pl.ds(m * N_BLK + HB, HB)
            return pltpu.make_async_remote_copy(
                src_ref=out_ref.at[rows_q2, sl],
                dst_ref=out_ref.at[rows_q2, sl],
                send_sem=yf_send.at[m], recv_sem=yf_recv.at[m],
                device_id=(mx, 1 - my, mz),
                device_id_type=pl.DeviceIdType.MESH,
            )

        def xf_descr(m):
            sl = pl.ds(m * N_BLK, HB)
            return pltpu.make_async_remote_copy(
                src_ref=out_ref.at[rows_q1, sl],
                dst_ref=out_ref.at[rows_q1, sl],
                send_sem=xf_send.at[m], recv_sem=xf_recv.at[m],
                device_id=(1 - mx, my, mz),
                device_id_type=pl.DeviceIdType.MESH,
            )

        def handle_z(m):
            z_descr(m).wait()
            sl = csl(m)
            c_ref[:, sl] = c_ref[:, sl] + zrecv_ref[:, sl]
            local_descr(m).start()
            xd_descr(m).start()
            yd_descr(m).start()

        def handle_xy(m):
            xd_descr(m).wait()
            yd_descr(m).wait()
            yf_descr(m).start()
            xf_descr(m).start()

        @pl.when(jnp.logical_and(n == 0, k == 0))
        def _barrier():
            bsem = pltpu.get_barrier_semaphore()
            for nbr in ((1 - mx, my, mz), (mx, 1 - my, mz), (mx, my, 1 - mz)):
                pl.semaphore_signal(bsem, inc=1, device_id=nbr,
                                    device_id_type=pl.DeviceIdType.MESH)
            pl.semaphore_wait(bsem, 3)

        acc = lax.dot_general(
            dy_ref[...], w_ref[...],
            (((1,), (1,)), ((), ())),
            preferred_element_type=jnp.float32,
        )
        nsl = csl(n)

        @pl.when(k == 0)
        def _init():
            c_ref[:, nsl] = acc

        @pl.when(k != 0)
        def _accum():
            c_ref[:, nsl] = c_ref[:, nsl] + acc

        @pl.when(k == K_STEPS - 1)
        def _comm():
            z_descr(n).start()

            @pl.when(n >= 1)
            def _():
                handle_z(n - 1)

            @pl.when(n >= 2)
            def _():
                handle_xy(n - 2)

            @pl.when(n == N_STEPS - 1)
            def _drain():
                handle_z(n)
                handle_xy(n - 1)
                handle_xy(n)
                for m in range(N_STEPS):
                    yf_descr(m).wait()
                    xf_descr(m).wait()
                    local_descr(m).wait()

    return pl.pallas_call(
        body,
        grid=(N_STEPS, K_STEPS),
        in_specs=[
            pl.BlockSpec((MQ, K_BLK), lambda n, k: (0, k)),
            pl.BlockSpec((N_BLK, K_BLK), lambda n, k: (n, k)),
        ],
        out_specs=pl.BlockSpec(memory_space=pl.ANY),
        out_shape=jax.ShapeDtypeStruct((M, N), jnp.float32),
        scratch_shapes=[
            pltpu.VMEM((MQ, N), jnp.float32),
            pltpu.VMEM((MQ, N), jnp.float32),
            pltpu.SemaphoreType.DMA((N_STEPS,)),
            pltpu.SemaphoreType.DMA((N_STEPS,)),
            pltpu.SemaphoreType.DMA((N_STEPS,)),
            pltpu.SemaphoreType.DMA((N_STEPS,)),
            pltpu.SemaphoreType.DMA((N_STEPS,)),
            pltpu.SemaphoreType.DMA((N_STEPS,)),
            pltpu.SemaphoreType.DMA((N_STEPS,)),
            pltpu.SemaphoreType.DMA((N_STEPS,)),
            pltpu.SemaphoreType.DMA((N_STEPS,)),
            pltpu.SemaphoreType.DMA((N_STEPS,)),
            pltpu.SemaphoreType.DMA((N_STEPS,)),
        ],
        compiler_params=pltpu.CompilerParams(
            collective_id=0,
            dimension_semantics=("arbitrary", "arbitrary"),
            vmem_limit_bytes=64 * 1024 * 1024,
        ),
    )(dy_q, W)


# device time: 364797 ns/iter; 2.5579x vs baseline; 1.0667x over previous
import jax
import jax.numpy as jnp
from jax import lax
from jax.experimental import pallas as pl
from jax.experimental.pallas import tpu as pltpu

M = 4096
K = 8192
N = 4096
MQ = M // 4
N_BLK = 512
K_BLK = 2048
N_STEPS = N // N_BLK
K_STEPS = K // K_BLK


def kernel(dy, W):
    my_x = lax.axis_index("x")
    my_y = lax.axis_index("y")
    q = 2 * my_x + my_y

    def body(q_ref, dy_ref, w_ref, out_ref, c_ref, zrecv_ref,
             local_sem, z_send, z_recv, xd_send, xd_recv,
             yd_send, yd_recv, yf_send, yf_recv, xf_send, xf_recv):
        n = pl.program_id(0)
        k = pl.program_id(1)
        mx = lax.axis_index("x")
        my = lax.axis_index("y")
        mz = lax.axis_index("z")
        qq = 2 * mx + my
        q1 = 2 * mx + (1 - my)
        q2 = 2 * (1 - mx) + my
        rows_q = pl.ds(qq * MQ, MQ)
        rows_q1 = pl.ds(q1 * MQ, MQ)
        rows_q2 = pl.ds(q2 * MQ, MQ)
        HB = N_BLK // 2

        def csl(m):
            return pl.ds(m * N_BLK, N_BLK)

        def z_descr(m):
            return pltpu.make_async_remote_copy(
                src_ref=c_ref.at[:, csl(m)], dst_ref=zrecv_ref.at[:, csl(m)],
                send_sem=z_send.at[m], recv_sem=z_recv.at[m],
                device_id=(mx, my, 1 - mz),
                device_id_type=pl.DeviceIdType.MESH,
            )

        def local_descr(m):
            return pltpu.make_async_copy(
                c_ref.at[:, csl(m)], out_ref.at[rows_q, csl(m)],
                local_sem.at[m],
            )

        def xd_descr(m):
            return pltpu.make_async_remote_copy(
                src_ref=c_ref.at[:, csl(m)], dst_ref=out_ref.at[rows_q, csl(m)],
                send_sem=xd_send.at[m], recv_sem=xd_recv.at[m],
                device_id=(1 - mx, my, mz),
                device_id_type=pl.DeviceIdType.MESH,
            )

        def yd_descr(m):
            return pltpu.make_async_remote_copy(
                src_ref=c_ref.at[:, csl(m)], dst_ref=out_ref.at[rows_q, csl(m)],
                send_sem=yd_send.at[m], recv_sem=yd_recv.at[m],
                device_id=(mx, 1 - my, mz),
                device_id_type=pl.DeviceIdType.MESH,
            )

        def yf_descr(m):
            sl = pl.ds(m * N_BLK + HB, HB)
            return pltpu.make_async_remote_copy(
                src_ref=out_ref.at[rows_q2, sl],
                dst_ref=out_ref.at[rows_q2, sl],
                send_sem=yf_send.at[m], recv_sem=yf_recv.at[m],
                device_id=(mx, 1 - my, mz),
                device_id_type=pl.DeviceIdType.MESH,
            )

        def xf_descr(m):
            sl = pl.ds(m * N_BLK, HB)
            return pltpu.make_async_remote_copy(
                src_ref=out_ref.at[rows_q1, sl],
                dst_ref=out_ref.at[rows_q1, sl],
                send_sem=xf_send.at[m], recv_sem=xf_recv.at[m],
                device_id=(1 - mx, my, mz),
                device_id_type=pl.DeviceIdType.MESH,
            )

        def handle_z(m):
            z_descr(m).wait()
            sl = csl(m)
            c_ref[:, sl] = c_ref[:, sl] + zrecv_ref[:, sl]
            local_descr(m).start()
            xd_descr(m).start()
            yd_descr(m).start()

        def handle_xy(m):
            xd_descr(m).wait()
            yd_descr(m).wait()
            yf_descr(m).start()
            xf_descr(m).start()

        @pl.when(jnp.logical_and(n == 0, k == 0))
        def _barrier():
            bsem = pltpu.get_barrier_semaphore()
            for nbr in ((1 - mx, my, mz), (mx, 1 - my, mz), (mx, my, 1 - mz)):
                pl.semaphore_signal(bsem, inc=1, device_id=nbr,
                                    device_id_type=pl.DeviceIdType.MESH)
            pl.semaphore_wait(bsem, 3)

        acc = lax.dot_general(
            dy_ref[...], w_ref[...],
            (((1,), (1,)), ((), ())),
            preferred_element_type=jnp.float32,
        )
        nsl = csl(n)

        @pl.when(k == 0)
        def _init():
            c_ref[:, nsl] = acc

        @pl.when(k != 0)
        def _accum():
            c_ref[:, nsl] = c_ref[:, nsl] + acc

        @pl.when(k == K_STEPS - 1)
        def _comm():
            z_descr(n).start()

            @pl.when(n >= 1)
            def _():
                handle_z(n - 1)

            @pl.when(n >= 2)
            def _():
                handle_xy(n - 2)

            @pl.when(n == N_STEPS - 1)
            def _drain():
                handle_z(n)
                handle_xy(n - 1)
                handle_xy(n)
                for m in range(N_STEPS):
                    yf_descr(m).wait()
                    xf_descr(m).wait()
                    local_descr(m).wait()

    grid_spec = pltpu.PrefetchScalarGridSpec(
        num_scalar_prefetch=1,
        grid=(N_STEPS, K_STEPS),
        in_specs=[
            pl.BlockSpec((MQ, K_BLK), lambda n, k, qs: (qs[0], k)),
            pl.BlockSpec((N_BLK, K_BLK), lambda n, k, qs: (n, k)),
        ],
        out_specs=pl.BlockSpec(memory_space=pl.ANY),
        scratch_shapes=[
            pltpu.VMEM((MQ, N), jnp.float32),
            pltpu.VMEM((MQ, N), jnp.float32),
            pltpu.SemaphoreType.DMA((N_STEPS,)),
            pltpu.SemaphoreType.DMA((N_STEPS,)),
            pltpu.SemaphoreType.DMA((N_STEPS,)),
            pltpu.SemaphoreType.DMA((N_STEPS,)),
            pltpu.SemaphoreType.DMA((N_STEPS,)),
            pltpu.SemaphoreType.DMA((N_STEPS,)),
            pltpu.SemaphoreType.DMA((N_STEPS,)),
            pltpu.SemaphoreType.DMA((N_STEPS,)),
            pltpu.SemaphoreType.DMA((N_STEPS,)),
            pltpu.SemaphoreType.DMA((N_STEPS,)),
            pltpu.SemaphoreType.DMA((N_STEPS,)),
        ],
    )

    return pl.pallas_call(
        body,
        grid_spec=grid_spec,
        out_shape=jax.ShapeDtypeStruct((M, N), jnp.float32),
        compiler_params=pltpu.CompilerParams(
            collective_id=0,
            dimension_semantics=("arbitrary", "arbitrary"),
            vmem_limit_bytes=64 * 1024 * 1024,
        ),
    )(q[None].astype(jnp.int32), dy, W)


# device time: 352700 ns/iter; 2.6456x vs baseline; 1.0343x over previous
import jax
import jax.numpy as jnp
from jax import lax
from jax.experimental import pallas as pl
from jax.experimental.pallas import tpu as pltpu

M = 4096
K = 8192
N = 4096
MQ = M // 4
N_BLK = 256
K_BLK = 2048
N_STEPS = N // N_BLK
K_STEPS = K // K_BLK


def kernel(dy, W):
    my_x = lax.axis_index("x")
    my_y = lax.axis_index("y")
    q = 2 * my_x + my_y

    def body(q_ref, dy_ref, w_ref, out_ref, c_ref, zrecv_ref,
             local_sem, z_send, z_recv, xd_send, xd_recv,
             yd_send, yd_recv, yf_send, yf_recv, xf_send, xf_recv):
        n = pl.program_id(0)
        k = pl.program_id(1)
        mx = lax.axis_index("x")
        my = lax.axis_index("y")
        mz = lax.axis_index("z")
        qq = 2 * mx + my
        q1 = 2 * mx + (1 - my)
        q2 = 2 * (1 - mx) + my
        rows_q = pl.ds(qq * MQ, MQ)
        rows_q1 = pl.ds(q1 * MQ, MQ)
        rows_q2 = pl.ds(q2 * MQ, MQ)
        HB = N_BLK // 2

        def csl(m):
            return pl.ds(m * N_BLK, N_BLK)

        def z_descr(m):
            return pltpu.make_async_remote_copy(
                src_ref=c_ref.at[:, csl(m)], dst_ref=zrecv_ref.at[:, csl(m)],
                send_sem=z_send.at[m], recv_sem=z_recv.at[m],
                device_id=(mx, my, 1 - mz),
                device_id_type=pl.DeviceIdType.MESH,
            )

        def local_descr(m):
            return pltpu.make_async_copy(
                c_ref.at[:, csl(m)], out_ref.at[rows_q, csl(m)],
                local_sem.at[m],
            )

        def xd_descr(m):
            return pltpu.make_async_remote_copy(
                src_ref=c_ref.at[:, csl(m)], dst_ref=out_ref.at[rows_q, csl(m)],
                send_sem=xd_send.at[m], recv_sem=xd_recv.at[m],
                device_id=(1 - mx, my, mz),
                device_id_type=pl.DeviceIdType.MESH,
            )

        def yd_descr(m):
            return pltpu.make_async_remote_copy(
                src_ref=c_ref.at[:, csl(m)], dst_ref=out_ref.at[rows_q, csl(m)],
                send_sem=yd_send.at[m], recv_sem=yd_recv.at[m],
                device_id=(mx, 1 - my, mz),
                device_id_type=pl.DeviceIdType.MESH,
            )

        def yf_descr(m):
            sl = pl.ds(m * N_BLK + HB, HB)
            return pltpu.make_async_remote_copy(
                src_ref=out_ref.at[rows_q2, sl],
                dst_ref=out_ref.at[rows_q2, sl],
                send_sem=yf_send.at[m], recv_sem=yf_recv.at[m],
                device_id=(mx, 1 - my, mz),
                device_id_type=pl.DeviceIdType.MESH,
            )

        def xf_descr(m):
            sl = pl.ds(m * N_BLK, HB)
            return pltpu.make_async_remote_copy(
                src_ref=out_ref.at[rows_q1, sl],
                dst_ref=out_ref.at[rows_q1, sl],
                send_sem=xf_send.at[m], recv_sem=xf_recv.at[m],
                device_id=(1 - mx, my, mz),
                device_id_type=pl.DeviceIdType.MESH,
            )

        def handle_z(m):
            z_descr(m).wait()
            sl = csl(m)
            c_ref[:, sl] = c_ref[:, sl] + zrecv_ref[:, sl]
            local_descr(m).start()
            xd_descr(m).start()
            yd_descr(m).start()

        def handle_xy(m):
            xd_descr(m).wait()
            yd_descr(m).wait()
            yf_descr(m).start()
            xf_descr(m).start()

        @pl.when(jnp.logical_and(n == 0, k == 0))
        def _barrier():
            bsem = pltpu.get_barrier_semaphore()
            for nbr in ((1 - mx, my, mz), (mx, 1 - my, mz), (mx, my, 1 - mz)):
                pl.semaphore_signal(bsem, inc=1, device_id=nbr,
                                    device_id_type=pl.DeviceIdType.MESH)
            pl.semaphore_wait(bsem, 3)

        acc = lax.dot_general(
            dy_ref[...], w_ref[...],
            (((1,), (1,)), ((), ())),
            preferred_element_type=jnp.float32,
        )
        nsl = csl(n)

        @pl.when(k == 0)
        def _init():
            c_ref[:, nsl] = acc

        @pl.when(k != 0)
        def _accum():
            c_ref[:, nsl] = c_ref[:, nsl] + acc

        @pl.when(k == K_STEPS - 1)
        def _comm():
            z_descr(n).start()

            @pl.when(n >= 1)
            def _():
                handle_z(n - 1)

            @pl.when(n >= 2)
            def _():
                handle_xy(n - 2)

            @pl.when(n == N_STEPS - 1)
            def _drain():
                handle_z(n)
                handle_xy(n - 1)
                handle_xy(n)
                for m in range(N_STEPS):
                    yf_descr(m).wait()
                    xf_descr(m).wait()
                    local_descr(m).wait()

    grid_spec = pltpu.PrefetchScalarGridSpec(
        num_scalar_prefetch=1,
        grid=(N_STEPS, K_STEPS),
        in_specs=[
            pl.BlockSpec((MQ, K_BLK), lambda n, k, qs: (qs[0], k)),
            pl.BlockSpec((N_BLK, K_BLK), lambda n, k, qs: (n, k)),
        ],
        out_specs=pl.BlockSpec(memory_space=pl.ANY),
        scratch_shapes=[
            pltpu.VMEM((MQ, N), jnp.float32),
            pltpu.VMEM((MQ, N), jnp.float32),
            pltpu.SemaphoreType.DMA((N_STEPS,)),
            pltpu.SemaphoreType.DMA((N_STEPS,)),
            pltpu.SemaphoreType.DMA((N_STEPS,)),
            pltpu.SemaphoreType.DMA((N_STEPS,)),
            pltpu.SemaphoreType.DMA((N_STEPS,)),
            pltpu.SemaphoreType.DMA((N_STEPS,)),
            pltpu.SemaphoreType.DMA((N_STEPS,)),
            pltpu.SemaphoreType.DMA((N_STEPS,)),
            pltpu.SemaphoreType.DMA((N_STEPS,)),
            pltpu.SemaphoreType.DMA((N_STEPS,)),
            pltpu.SemaphoreType.DMA((N_STEPS,)),
        ],
    )

    return pl.pallas_call(
        body,
        grid_spec=grid_spec,
        out_shape=jax.ShapeDtypeStruct((M, N), jnp.float32),
        compiler_params=pltpu.CompilerParams(
            collective_id=0,
            dimension_semantics=("arbitrary", "arbitrary"),
            vmem_limit_bytes=64 * 1024 * 1024,
        ),
    )(q[None].astype(jnp.int32), dy, W)


# device time: 349400 ns/iter; 2.6706x vs baseline; 1.0094x over previous
import jax
import jax.numpy as jnp
from jax import lax
from jax.experimental import pallas as pl
from jax.experimental.pallas import tpu as pltpu

M = 4096
K = 8192
N = 4096
MQ = M // 4
N_BLK = 256
K_BLK = 2048
N_STEPS = N // N_BLK
K_STEPS = K // K_BLK


def kernel(dy, W):
    my_x = lax.axis_index("x")
    my_y = lax.axis_index("y")
    q = 2 * my_x + my_y

    def body(q_ref, dy_ref, w_ref, out_ref, c_ref, zrecv_ref,
             local_sem, z_send, z_recv, xd_send, xd_recv,
             yd_send, yd_recv, yf_send, yf_recv, xf_send, xf_recv,
             zf_send, zf_recv):
        n = pl.program_id(0)
        k = pl.program_id(1)
        mx = lax.axis_index("x")
        my = lax.axis_index("y")
        mz = lax.axis_index("z")
        qq = 2 * mx + my
        q1 = 2 * mx + (1 - my)
        q2 = 2 * (1 - mx) + my
        q3 = 2 * (1 - mx) + (1 - my)
        rows_q = pl.ds(qq * MQ, MQ)
        rows_q1 = pl.ds(q1 * MQ, MQ)
        rows_q2 = pl.ds(q2 * MQ, MQ)
        rows_q3 = pl.ds(q3 * MQ, MQ)
        HF = N_BLK // 2
        yf_off = mz * HF

        def csl(m):
            return pl.ds(m * N_BLK, N_BLK)

        def z_descr(m):
            return pltpu.make_async_remote_copy(
                src_ref=c_ref.at[:, csl(m)], dst_ref=zrecv_ref.at[:, csl(m)],
                send_sem=z_send.at[m], recv_sem=z_recv.at[m],
                device_id=(mx, my, 1 - mz),
                device_id_type=pl.DeviceIdType.MESH,
            )

        def local_descr(m):
            return pltpu.make_async_copy(
                c_ref.at[:, csl(m)], out_ref.at[rows_q, csl(m)],
                local_sem.at[m],
            )

        def xd_descr(m):
            return pltpu.make_async_remote_copy(
                src_ref=c_ref.at[:, csl(m)], dst_ref=out_ref.at[rows_q, csl(m)],
                send_sem=xd_send.at[m], recv_sem=xd_recv.at[m],
                device_id=(1 - mx, my, mz),
                device_id_type=pl.DeviceIdType.MESH,
            )

        def yd_descr(m):
            return pltpu.make_async_remote_copy(
                src_ref=c_ref.at[:, csl(m)], dst_ref=out_ref.at[rows_q, csl(m)],
                send_sem=yd_send.at[m], recv_sem=yd_recv.at[m],
                device_id=(mx, 1 - my, mz),
                device_id_type=pl.DeviceIdType.MESH,
            )

        def yf_descr(m):
            sl = pl.ds(m * N_BLK + yf_off, HF)
            return pltpu.make_async_remote_copy(
                src_ref=out_ref.at[rows_q2, sl],
                dst_ref=out_ref.at[rows_q2, sl],
                send_sem=yf_send.at[m], recv_sem=yf_recv.at[m],
                device_id=(mx, 1 - my, mz),
                device_id_type=pl.DeviceIdType.MESH,
            )

        def xf_descr(m):
            sl = csl(m)
            return pltpu.make_async_remote_copy(
                src_ref=out_ref.at[rows_q1, sl],
                dst_ref=out_ref.at[rows_q1, sl],
                send_sem=xf_send.at[m], recv_sem=xf_recv.at[m],
                device_id=(1 - mx, my, mz),
                device_id_type=pl.DeviceIdType.MESH,
            )

        def zf_descr(m):
            sl = pl.ds(m * N_BLK + yf_off, HF)
            return pltpu.make_async_remote_copy(
                src_ref=out_ref.at[rows_q3, sl],
                dst_ref=out_ref.at[rows_q3, sl],
                send_sem=zf_send.at[m], recv_sem=zf_recv.at[m],
                device_id=(mx, my, 1 - mz),
                device_id_type=pl.DeviceIdType.MESH,
            )

        def handle_z(m):
            z_descr(m).wait()
            sl = csl(m)
            c_ref[:, sl] = c_ref[:, sl] + zrecv_ref[:, sl]
            local_descr(m).start()
            xd_descr(m).start()
            yd_descr(m).start()

        def handle_xy(m):
            xd_descr(m).wait()
            yd_descr(m).wait()

            @pl.when(m % 3 == 2)
            def _():
                xf_descr(m).start()

            @pl.when(m % 3 != 2)
            def _():
                yf_descr(m).start()

        def handle_zf(m):
            @pl.when(m % 3 != 2)
            def _():
                yf_descr(m).wait()
                zf_descr(m).start()

        @pl.when(jnp.logical_and(n == 0, k == 0))
        def _barrier():
            bsem = pltpu.get_barrier_semaphore()
            for nbr in ((1 - mx, my, mz), (mx, 1 - my, mz), (mx, my, 1 - mz)):
                pl.semaphore_signal(bsem, inc=1, device_id=nbr,
                                    device_id_type=pl.DeviceIdType.MESH)
            pl.semaphore_wait(bsem, 3)

        acc = lax.dot_general(
            dy_ref[...], w_ref[...],
            (((1,), (1,)), ((), ())),
            preferred_element_type=jnp.float32,
        )
        nsl = csl(n)

        @pl.when(k == 0)
        def _init():
            c_ref[:, nsl] = acc

        @pl.when(k != 0)
        def _accum():
            c_ref[:, nsl] = c_ref[:, nsl] + acc

        @pl.when(k == K_STEPS - 1)
        def _comm():
            z_descr(n).start()

            @pl.when(n >= 1)
            def _():
                handle_z(n - 1)

            @pl.when(n >= 2)
            def _():
                handle_xy(n - 2)

            @pl.when(n >= 3)
            def _():
                handle_zf(n - 3)

            @pl.when(n == N_STEPS - 1)
            def _drain():
                handle_z(n)
                handle_xy(n - 1)
                handle_xy(n)
                handle_zf(n - 2)
                handle_zf(n - 1)
                handle_zf(n)
                for m in range(N_STEPS):
                    if m % 3 == 2:
                        xf_descr(m).wait()
                    else:
                        zf_descr(m).wait()
                    local_descr(m).wait()

    grid_spec = pltpu.PrefetchScalarGridSpec(
        num_scalar_prefetch=1,
        grid=(N_STEPS, K_STEPS),
        in_specs=[
            pl.BlockSpec((MQ, K_BLK), lambda n, k, qs: (qs[0], k)),
            pl.BlockSpec((N_BLK, K_BLK), lambda n, k, qs: (n, k)),
        ],
        out_specs=pl.BlockSpec(memory_space=pl.ANY),
        scratch_shapes=[
            pltpu.VMEM((MQ, N), jnp.float32),
            pltpu.VMEM((MQ, N), jnp.float32),
            pltpu.SemaphoreType.DMA((N_STEPS,)),
            pltpu.SemaphoreType.DMA((N_STEPS,)),
            pltpu.SemaphoreType.DMA((N_STEPS,)),
            pltpu.SemaphoreType.DMA((N_STEPS,)),
            pltpu.SemaphoreType.DMA((N_STEPS,)),
            pltpu.SemaphoreType.DMA((N_STEPS,)),
            pltpu.SemaphoreType.DMA((N_STEPS,)),
            pltpu.SemaphoreType.DMA((N_STEPS,)),
            pltpu.SemaphoreType.DMA((N_STEPS,)),
            pltpu.SemaphoreType.DMA((N_STEPS,)),
            pltpu.SemaphoreType.DMA((N_STEPS,)),
            pltpu.SemaphoreType.DMA((N_STEPS,)),
            pltpu.SemaphoreType.DMA((N_STEPS,)),
        ],
    )

    return pl.pallas_call(
        body,
        grid_spec=grid_spec,
        out_shape=jax.ShapeDtypeStruct((M, N), jnp.float32),
        compiler_params=pltpu.CompilerParams(
            collective_id=0,
            dimension_semantics=("arbitrary", "arbitrary"),
            vmem_limit_bytes=64 * 1024 * 1024,
        ),
    )(q[None].astype(jnp.int32), dy, W)
